# baseline (speedup 1.0000x reference)
import functools

import jax
import jax.numpy as jnp
from jax import lax
from jax.experimental import pallas as pl
from jax.experimental.pallas import tpu as pltpu
from jax.experimental.pallas import tpu_sc as plsc

E = 8
D_MODEL = 2048
D_FF = 768
FF_B = 256
BLOCK = 128
T_FIXED = 2048
NB = (2 * T_FIXED + E * (BLOCK - 1)) // BLOCK + 1
P = NB * BLOCK
NW = 32
TPW = 2 * T_FIXED // NW // 2
CH = 16


def _router_body(x_ref, wg_ref, logits_ref, pos1_ref, pos2_ref,
                 wrep1_ref, wrep2_ref, bexp_ref, nblk_ref, *, n_tok):
    x = x_ref[...]
    logits = lax.dot_general(x, wg_ref[...], (((1,), (1,)), ((), ())),
                             preferred_element_type=jnp.float32)
    logits_ref[...] = logits
    idx = lax.broadcasted_iota(jnp.int32, (n_tok, E), 1)
    m1 = jnp.max(logits, axis=1, keepdims=True)
    i1 = -jnp.max(jnp.where(logits == m1, -idx, -E - 1), axis=1,
                  keepdims=True)
    masked = jnp.where(idx == i1, -jnp.inf, logits)
    m2 = jnp.max(masked, axis=1, keepdims=True)
    i2 = -jnp.max(jnp.where(masked == m2, -idx, -E - 1), axis=1,
                  keepdims=True)
    w1 = 1.0 / (1.0 + jnp.exp(m2 - m1))
    w2 = 1.0 - w1
    oh1 = (idx == i1).astype(jnp.float32)
    oh2 = (idx == i2).astype(jnp.float32)

    rr = lax.broadcasted_iota(jnp.int32, (n_tok, n_tok), 0)
    cc = lax.broadcasted_iota(jnp.int32, (n_tok, n_tok), 1)
    ltri = (cc <= rr).astype(jnp.bfloat16)
    csum1 = lax.dot_general(ltri, oh1.astype(jnp.bfloat16),
                            (((1,), (0,)), ((), ())),
                            preferred_element_type=jnp.float32)
    csum2 = lax.dot_general(ltri, oh2.astype(jnp.bfloat16),
                            (((1,), (0,)), ((), ())),
                            preferred_element_type=jnp.float32)
    cnt1 = csum1[n_tok - 1:n_tok, :]
    cnt2 = csum2[n_tok - 1:n_tok, :]
    cnt = cnt1 + cnt2
    padded = jnp.floor((cnt + (BLOCK - 1)) / BLOCK) * BLOCK
    er = lax.broadcasted_iota(jnp.int32, (E, E), 0)
    ec = lax.broadcasted_iota(jnp.int32, (E, E), 1)
    slt = (er < ec).astype(jnp.float32)
    offs = lax.dot_general(padded, slt, (((1,), (0,)), ((), ())),
                           preferred_element_type=jnp.float32)

    pos1 = jnp.sum(oh1 * (offs + csum1 - 1.0), axis=1, keepdims=True)
    pos2 = jnp.sum(oh2 * (offs + cnt1 + csum2 - 1.0), axis=1,
                   keepdims=True)
    pos1_ref[...] = pos1.astype(jnp.int32)
    pos2_ref[...] = pos2.astype(jnp.int32)
    wrep1_ref[...] = lax.broadcast_in_dim(w1, (n_tok, 128), (0, 1))
    wrep2_ref[...] = lax.broadcast_in_dim(w2, (n_tok, 128), (0, 1))

    bi = lax.broadcasted_iota(jnp.int32, (NB, E), 0).astype(jnp.float32)
    ei = lax.broadcasted_iota(jnp.int32, (NB, E), 1)
    bstart = bi * BLOCK
    lo = lax.broadcast_in_dim(offs, (NB, E), (0, 1))
    hi = lax.broadcast_in_dim(offs + padded, (NB, E), (0, 1))
    ind = (bstart >= lo - 0.5) & (bstart < hi - 0.5)
    be = 7 - jnp.sum(jnp.where(ind, 7 - ei, 0), axis=1, keepdims=True)
    bexp_ref[...] = be.astype(jnp.int32)
    nblk_ref[...] = (jnp.sum(padded, axis=1, keepdims=True)
                     / BLOCK).astype(jnp.int32)


def _scatter_body(x_hbm, pos1_hbm, pos2_hbm, wrep1_hbm, wrep2_hbm,
                  xs_hbm, wrow_hbm, rb0, rb1, wb1, wb2,
                  idx1_v, idx2_v, sem_in, sem):
    wid = lax.axis_index("s") * 2 + lax.axis_index("c")
    base = wid * TPW
    nch = TPW // CH
    pltpu.sync_copy(pos1_hbm.at[pl.ds(base, TPW)], idx1_v)
    pltpu.sync_copy(pos2_hbm.at[pl.ds(base, TPW)], idx2_v)
    pltpu.sync_copy(wrep1_hbm.at[pl.ds(base, TPW), :], wb1)
    pltpu.sync_copy(wrep2_hbm.at[pl.ds(base, TPW), :], wb2)
    cp_in = pltpu.async_copy(x_hbm.at[pl.ds(base, CH), :], rb0, sem_in)
    for c in range(nch):
        cur = rb0 if c % 2 == 0 else rb1
        nxt = rb1 if c % 2 == 0 else rb0
        cp_in.wait()
        if c + 1 < nch:
            cp_in = pltpu.async_copy(
                x_hbm.at[pl.ds(base + CH * (c + 1), CH), :], nxt, sem_in)
        i1v = idx1_v[pl.ds(CH * c, CH)]
        i2v = idx2_v[pl.ds(CH * c, CH)]
        s1 = pltpu.async_copy(cur, xs_hbm.at[i1v], sem)
        s2 = pltpu.async_copy(cur, xs_hbm.at[i2v], sem)
        s3 = pltpu.async_copy(wb1.at[pl.ds(CH * c, CH), :],
                              wrow_hbm.at[i1v], sem)
        s4 = pltpu.async_copy(wb2.at[pl.ds(CH * c, CH), :],
                              wrow_hbm.at[i2v], sem)
        s1.wait()
        s2.wait()
        s3.wait()
        s4.wait()


def _group_body(bexp_ref, nblk_ref, xs_ref, wrow_ref,
                wgate_ref, wup_ref, wdown_ref, ys_ref):
    b = pl.program_id(0)

    @pl.when(b < nblk_ref[0])
    def _():
        xt = xs_ref[...]
        g = lax.dot_general(xt, wgate_ref[0], (((1,), (1,)), ((), ())),
                            preferred_element_type=jnp.float32)
        u = lax.dot_general(xt, wup_ref[0], (((1,), (1,)), ((), ())),
                            preferred_element_type=jnp.float32)
        h = (g * lax.logistic(g)) * u
        y = lax.dot_general(h, wdown_ref[0], (((1,), (1,)), ((), ())),
                            preferred_element_type=jnp.float32)
        wcol = wrow_ref[:, 0:1]
        ys_ref[...] = wcol * y


def _gather_body(ys_hbm, pos1_hbm, pos2_hbm, out_hbm,
                 buf1, buf2, idx1_v, idx2_v, sem):
    wid = lax.axis_index("s") * 2 + lax.axis_index("c")
    base = wid * TPW
    pltpu.sync_copy(pos1_hbm.at[pl.ds(base, TPW)], idx1_v)
    pltpu.sync_copy(pos2_hbm.at[pl.ds(base, TPW)], idx2_v)
    for c in range(TPW // CH):
        rb = base + CH * c
        i1v = idx1_v[pl.ds(CH * c, CH)]
        i2v = idx2_v[pl.ds(CH * c, CH)]
        cp1 = pltpu.async_copy(ys_hbm.at[i1v], buf1, sem)
        cp2 = pltpu.async_copy(ys_hbm.at[i2v], buf2, sem)
        cp1.wait()
        cp2.wait()

        def _acc(i, carry):
            for j in range(CH):
                buf1[j, pl.ds(i * 16, 16)] = (buf1[j, pl.ds(i * 16, 16)]
                                              + buf2[j, pl.ds(i * 16, 16)])
            return carry

        lax.fori_loop(0, D_MODEL // 16, _acc, 0)
        pltpu.sync_copy(buf1, out_hbm.at[pl.ds(rb, CH), :])


def kernel(hidden_states, Wg, W_gate, W_up, W_down):
    B, S, D = hidden_states.shape
    x = hidden_states.reshape(-1, D)
    T = x.shape[0]

    logits, pos1, pos2, wrep1, wrep2, bexp, nblk = pl.pallas_call(
        functools.partial(_router_body, n_tok=T),
        out_shape=[
            jax.ShapeDtypeStruct((T, E), jnp.float32),
            jax.ShapeDtypeStruct((T, 1), jnp.int32),
            jax.ShapeDtypeStruct((T, 1), jnp.int32),
            jax.ShapeDtypeStruct((T, 128), jnp.float32),
            jax.ShapeDtypeStruct((T, 128), jnp.float32),
            jax.ShapeDtypeStruct((NB, 1), jnp.int32),
            jax.ShapeDtypeStruct((1, 1), jnp.int32),
        ],
    )(x, Wg)

    pos1f = pos1.reshape(T)
    pos2f = pos2.reshape(T)

    mesh = plsc.VectorSubcoreMesh(core_axis_name="c", subcore_axis_name="s")
    xs, wrow = pl.kernel(
        _scatter_body,
        out_type=[
            jax.ShapeDtypeStruct((P, D), jnp.float32),
            jax.ShapeDtypeStruct((P, 128), jnp.float32),
        ],
        mesh=mesh,
        scratch_types=[
            pltpu.VMEM((CH, D), jnp.float32),
            pltpu.VMEM((CH, D), jnp.float32),
            pltpu.VMEM((TPW, 128), jnp.float32),
            pltpu.VMEM((TPW, 128), jnp.float32),
            pltpu.VMEM((TPW,), jnp.int32),
            pltpu.VMEM((TPW,), jnp.int32),
            pltpu.SemaphoreType.DMA,
            pltpu.SemaphoreType.DMA,
        ],
    )(x, pos1f, pos2f, wrep1, wrep2)

    grid_spec = pltpu.PrefetchScalarGridSpec(
        num_scalar_prefetch=2,
        grid=(NB,),
        in_specs=[
            pl.BlockSpec((BLOCK, D),
                         lambda b, be, nb: (jnp.minimum(b, nb[0] - 1), 0)),
            pl.BlockSpec((BLOCK, 128),
                         lambda b, be, nb: (jnp.minimum(b, nb[0] - 1), 0)),
            pl.BlockSpec((1, D_FF, D), lambda b, be, nb: (be[b], 0, 0)),
            pl.BlockSpec((1, D_FF, D), lambda b, be, nb: (be[b], 0, 0)),
            pl.BlockSpec((1, D, D_FF), lambda b, be, nb: (be[b], 0, 0)),
        ],
        out_specs=pl.BlockSpec(
            (BLOCK, D), lambda b, be, nb: (jnp.minimum(b, nb[0] - 1), 0)),
    )
    ys = pl.pallas_call(
        _group_body,
        grid_spec=grid_spec,
        out_shape=jax.ShapeDtypeStruct((P, D), jnp.float32),
    )(bexp.reshape(NB), nblk.reshape(1), xs, wrow, W_gate, W_up, W_down)

    out = pl.kernel(
        _gather_body,
        out_type=jax.ShapeDtypeStruct((T, D), jnp.float32),
        mesh=mesh,
        scratch_types=[
            pltpu.VMEM((CH, D), jnp.float32),
            pltpu.VMEM((CH, D), jnp.float32),
            pltpu.VMEM((TPW,), jnp.int32),
            pltpu.VMEM((TPW,), jnp.int32),
            pltpu.SemaphoreType.DMA,
        ],
    )(ys, pos1f, pos2f)

    return out.reshape(B, S, D), logits


# speedup vs baseline: 1.2803x; 1.2803x over previous
import functools

import jax
import jax.numpy as jnp
from jax import lax
from jax.experimental import pallas as pl
from jax.experimental.pallas import tpu as pltpu
from jax.experimental.pallas import tpu_sc as plsc

E = 8
D_MODEL = 2048
D_FF = 768
FF_B = 256
BLOCK = 256
T_FIXED = 2048
NB = (2 * T_FIXED + E * (BLOCK - 1)) // BLOCK + 1
P = NB * BLOCK
NW = 32
TPW = 2 * T_FIXED // NW // 2
CH = 16


def _router_body(x_ref, wg_ref, logits_ref, pos1_ref, pos2_ref,
                 wrep1_ref, wrep2_ref, bexp_ref, nblk_ref, *, n_tok):
    x = x_ref[...]
    logits = lax.dot_general(x, wg_ref[...], (((1,), (1,)), ((), ())),
                             preferred_element_type=jnp.float32)
    logits_ref[...] = logits
    idx = lax.broadcasted_iota(jnp.int32, (n_tok, E), 1)
    m1 = jnp.max(logits, axis=1, keepdims=True)
    i1 = -jnp.max(jnp.where(logits == m1, -idx, -E - 1), axis=1,
                  keepdims=True)
    masked = jnp.where(idx == i1, -jnp.inf, logits)
    m2 = jnp.max(masked, axis=1, keepdims=True)
    i2 = -jnp.max(jnp.where(masked == m2, -idx, -E - 1), axis=1,
                  keepdims=True)
    w1 = 1.0 / (1.0 + jnp.exp(m2 - m1))
    w2 = 1.0 - w1
    oh1 = (idx == i1).astype(jnp.float32)
    oh2 = (idx == i2).astype(jnp.float32)

    rr = lax.broadcasted_iota(jnp.int32, (n_tok, n_tok), 0)
    cc = lax.broadcasted_iota(jnp.int32, (n_tok, n_tok), 1)
    ltri = (cc <= rr).astype(jnp.bfloat16)
    csum1 = lax.dot_general(ltri, oh1.astype(jnp.bfloat16),
                            (((1,), (0,)), ((), ())),
                            preferred_element_type=jnp.float32)
    csum2 = lax.dot_general(ltri, oh2.astype(jnp.bfloat16),
                            (((1,), (0,)), ((), ())),
                            preferred_element_type=jnp.float32)
    cnt1 = csum1[n_tok - 1:n_tok, :]
    cnt2 = csum2[n_tok - 1:n_tok, :]
    cnt = cnt1 + cnt2
    padded = jnp.floor((cnt + (BLOCK - 1)) / BLOCK) * BLOCK
    er = lax.broadcasted_iota(jnp.int32, (E, E), 0)
    ec = lax.broadcasted_iota(jnp.int32, (E, E), 1)
    slt = (er < ec).astype(jnp.float32)
    offs = lax.dot_general(padded, slt, (((1,), (0,)), ((), ())),
                           preferred_element_type=jnp.float32)

    pos1 = jnp.sum(oh1 * (offs + csum1 - 1.0), axis=1, keepdims=True)
    pos2 = jnp.sum(oh2 * (offs + cnt1 + csum2 - 1.0), axis=1,
                   keepdims=True)
    pos1_ref[...] = pos1.astype(jnp.int32)
    pos2_ref[...] = pos2.astype(jnp.int32)
    wrep1_ref[...] = lax.broadcast_in_dim(w1, (n_tok, 128), (0, 1))
    wrep2_ref[...] = lax.broadcast_in_dim(w2, (n_tok, 128), (0, 1))

    bi = lax.broadcasted_iota(jnp.int32, (NB, E), 0).astype(jnp.float32)
    ei = lax.broadcasted_iota(jnp.int32, (NB, E), 1)
    bstart = bi * BLOCK
    lo = lax.broadcast_in_dim(offs, (NB, E), (0, 1))
    hi = lax.broadcast_in_dim(offs + padded, (NB, E), (0, 1))
    ind = (bstart >= lo - 0.5) & (bstart < hi - 0.5)
    be = 7 - jnp.sum(jnp.where(ind, 7 - ei, 0), axis=1, keepdims=True)
    bexp_ref[...] = be.astype(jnp.int32)
    nblk_ref[...] = (jnp.sum(padded, axis=1, keepdims=True)
                     / BLOCK).astype(jnp.int32)


def _scatter_body(x_hbm, pos1_hbm, pos2_hbm, wrep1_hbm, wrep2_hbm,
                  xs_hbm, wrow_hbm, rb0, rb1, wb1, wb2,
                  idx1_v, idx2_v, sem_in, sem):
    wid = lax.axis_index("s") * 2 + lax.axis_index("c")
    base = wid * TPW
    nch = TPW // CH
    pltpu.sync_copy(pos1_hbm.at[pl.ds(base, TPW)], idx1_v)
    pltpu.sync_copy(pos2_hbm.at[pl.ds(base, TPW)], idx2_v)
    pltpu.sync_copy(wrep1_hbm.at[pl.ds(base, TPW), :], wb1)
    pltpu.sync_copy(wrep2_hbm.at[pl.ds(base, TPW), :], wb2)
    cp_in = pltpu.async_copy(x_hbm.at[pl.ds(base, CH), :], rb0, sem_in)
    for c in range(nch):
        cur = rb0 if c % 2 == 0 else rb1
        nxt = rb1 if c % 2 == 0 else rb0
        cp_in.wait()
        if c + 1 < nch:
            cp_in = pltpu.async_copy(
                x_hbm.at[pl.ds(base + CH * (c + 1), CH), :], nxt, sem_in)
        i1v = idx1_v[pl.ds(CH * c, CH)]
        i2v = idx2_v[pl.ds(CH * c, CH)]
        s1 = pltpu.async_copy(cur, xs_hbm.at[i1v], sem)
        s2 = pltpu.async_copy(cur, xs_hbm.at[i2v], sem)
        s3 = pltpu.async_copy(wb1.at[pl.ds(CH * c, CH), :],
                              wrow_hbm.at[i1v], sem)
        s4 = pltpu.async_copy(wb2.at[pl.ds(CH * c, CH), :],
                              wrow_hbm.at[i2v], sem)
        s1.wait()
        s2.wait()
        s3.wait()
        s4.wait()


def _group_body(bexp_ref, nblk_ref, xs_ref, wrow_ref,
                wgate_ref, wup_ref, wdown_ref, ys_ref):
    b = pl.program_id(0)

    @pl.when(b < nblk_ref[0])
    def _():
        xt = xs_ref[...]
        g = lax.dot_general(xt, wgate_ref[0], (((1,), (1,)), ((), ())),
                            preferred_element_type=jnp.float32)
        u = lax.dot_general(xt, wup_ref[0], (((1,), (1,)), ((), ())),
                            preferred_element_type=jnp.float32)
        h = (g * lax.logistic(g)) * u
        y = lax.dot_general(h, wdown_ref[0], (((1,), (1,)), ((), ())),
                            preferred_element_type=jnp.float32)
        wcol = wrow_ref[:, 0:1]
        ys_ref[...] = wcol * y


def _gather_body(ys_hbm, pos1_hbm, pos2_hbm, out_hbm,
                 buf1, buf2, idx1_v, idx2_v, sem):
    wid = lax.axis_index("s") * 2 + lax.axis_index("c")
    base = wid * TPW
    pltpu.sync_copy(pos1_hbm.at[pl.ds(base, TPW)], idx1_v)
    pltpu.sync_copy(pos2_hbm.at[pl.ds(base, TPW)], idx2_v)
    for c in range(TPW // CH):
        rb = base + CH * c
        i1v = idx1_v[pl.ds(CH * c, CH)]
        i2v = idx2_v[pl.ds(CH * c, CH)]
        cp1 = pltpu.async_copy(ys_hbm.at[i1v], buf1, sem)
        cp2 = pltpu.async_copy(ys_hbm.at[i2v], buf2, sem)
        cp1.wait()
        cp2.wait()

        def _acc(i, carry):
            for j in range(CH):
                buf1[j, pl.ds(i * 16, 16)] = (buf1[j, pl.ds(i * 16, 16)]
                                              + buf2[j, pl.ds(i * 16, 16)])
            return carry

        lax.fori_loop(0, D_MODEL // 16, _acc, 0)
        pltpu.sync_copy(buf1, out_hbm.at[pl.ds(rb, CH), :])


def kernel(hidden_states, Wg, W_gate, W_up, W_down):
    B, S, D = hidden_states.shape
    x = hidden_states.reshape(-1, D)
    T = x.shape[0]

    logits, pos1, pos2, wrep1, wrep2, bexp, nblk = pl.pallas_call(
        functools.partial(_router_body, n_tok=T),
        out_shape=[
            jax.ShapeDtypeStruct((T, E), jnp.float32),
            jax.ShapeDtypeStruct((T, 1), jnp.int32),
            jax.ShapeDtypeStruct((T, 1), jnp.int32),
            jax.ShapeDtypeStruct((T, 128), jnp.float32),
            jax.ShapeDtypeStruct((T, 128), jnp.float32),
            jax.ShapeDtypeStruct((NB, 1), jnp.int32),
            jax.ShapeDtypeStruct((1, 1), jnp.int32),
        ],
    )(x, Wg)

    pos1f = pos1.reshape(T)
    pos2f = pos2.reshape(T)

    mesh = plsc.VectorSubcoreMesh(core_axis_name="c", subcore_axis_name="s")
    xs, wrow = pl.kernel(
        _scatter_body,
        out_type=[
            jax.ShapeDtypeStruct((P, D), jnp.float32),
            jax.ShapeDtypeStruct((P, 128), jnp.float32),
        ],
        mesh=mesh,
        scratch_types=[
            pltpu.VMEM((CH, D), jnp.float32),
            pltpu.VMEM((CH, D), jnp.float32),
            pltpu.VMEM((TPW, 128), jnp.float32),
            pltpu.VMEM((TPW, 128), jnp.float32),
            pltpu.VMEM((TPW,), jnp.int32),
            pltpu.VMEM((TPW,), jnp.int32),
            pltpu.SemaphoreType.DMA,
            pltpu.SemaphoreType.DMA,
        ],
    )(x, pos1f, pos2f, wrep1, wrep2)

    grid_spec = pltpu.PrefetchScalarGridSpec(
        num_scalar_prefetch=2,
        grid=(NB,),
        in_specs=[
            pl.BlockSpec((BLOCK, D),
                         lambda b, be, nb: (jnp.minimum(b, nb[0] - 1), 0)),
            pl.BlockSpec((BLOCK, 128),
                         lambda b, be, nb: (jnp.minimum(b, nb[0] - 1), 0)),
            pl.BlockSpec((1, D_FF, D), lambda b, be, nb: (be[b], 0, 0)),
            pl.BlockSpec((1, D_FF, D), lambda b, be, nb: (be[b], 0, 0)),
            pl.BlockSpec((1, D, D_FF), lambda b, be, nb: (be[b], 0, 0)),
        ],
        out_specs=pl.BlockSpec(
            (BLOCK, D), lambda b, be, nb: (jnp.minimum(b, nb[0] - 1), 0)),
    )
    ys = pl.pallas_call(
        _group_body,
        grid_spec=grid_spec,
        out_shape=jax.ShapeDtypeStruct((P, D), jnp.float32),
    )(bexp.reshape(NB), nblk.reshape(1), xs, wrow, W_gate, W_up, W_down)

    out = pl.kernel(
        _gather_body,
        out_type=jax.ShapeDtypeStruct((T, D), jnp.float32),
        mesh=mesh,
        scratch_types=[
            pltpu.VMEM((CH, D), jnp.float32),
            pltpu.VMEM((CH, D), jnp.float32),
            pltpu.VMEM((TPW,), jnp.int32),
            pltpu.VMEM((TPW,), jnp.int32),
            pltpu.SemaphoreType.DMA,
        ],
    )(ys, pos1f, pos2f)

    return out.reshape(B, S, D), logits
